# Initial kernel scaffold; baseline (speedup 1.0000x reference)
import jax
import jax.numpy as jnp
from jax.experimental import pallas as pl


def kernel(x, batch):
    raise NotImplementedError("write your pallas kernel here")


# speedup vs baseline: 2.9150x; 2.9150x over previous
import functools

import jax
import jax.numpy as jnp
from jax import lax
from jax.experimental import pallas as pl
from jax.experimental.pallas import tpu as pltpu
from jax.experimental.pallas import tpu_sc as plsc

NC = 2
NS = 16
NW = NC * NS
L = 16
NUM_SEG = 64
SEG_PER_W = NUM_SEG // NW
N_ROWS = 50000
D = 256
NJ = D // L
CH = 128


def _body(x_hbm, bounds_hbm, out_hbm, bounds_v, buf_v, acc_v):
    cid = lax.axis_index("c")
    sid = lax.axis_index("s")
    wid = sid * NC + cid

    pltpu.sync_copy(bounds_hbm, bounds_v)

    bv = bounds_v[pl.ds(SEG_PER_W * wid, L)]
    a0 = bv[0]
    m = bv[1]
    b1 = bv[2]

    zero = jnp.zeros((L,), jnp.float32)
    for j in range(SEG_PER_W * NJ):
        acc_v[pl.ds(j * L, L)] = zero

    a8 = lax.div(a0, 8) * 8
    total = b1 - a8
    nch = lax.div(total + (CH - 1), CH)

    def chunk_body(c, carry):
        base = a8 + c * CH
        clamped = pl.multiple_of(jnp.minimum(base, N_ROWS - CH), 8)
        pltpu.sync_copy(x_hbm.at[pl.ds(clamped, CH)], buf_v)
        lo0 = jnp.maximum(base, a0) - clamped
        hi0 = jnp.maximum(jnp.minimum(m, clamped + CH) - clamped, lo0)
        lo1 = jnp.maximum(base, m) - clamped
        hi1 = jnp.maximum(jnp.minimum(b1, clamped + CH) - clamped, lo1)

        def make_row_body(s):
            def row_body(r, rc):
                for j in range(NJ):
                    plsc.addupdate(acc_v.at[pl.ds(s * D + j * L, L)],
                                   buf_v[r, pl.ds(j * L, L)])
                return rc
            return row_body

        lax.fori_loop(lo0, hi0, make_row_body(0), 0)
        lax.fori_loop(lo1, hi1, make_row_body(1), 0)
        return carry

    lax.fori_loop(0, nch, chunk_body, 0)

    one = jnp.ones((L,), jnp.float32)
    n0 = one * (m - a0).astype(jnp.float32)
    n1 = one * (b1 - m).astype(jnp.float32)
    s0 = jnp.where(n0 > 0.0, one / jnp.maximum(n0, one), 0.0)
    s1 = jnp.where(n1 > 0.0, one / jnp.maximum(n1, one), 0.0)
    for j in range(NJ):
        acc_v[pl.ds(j * L, L)] = acc_v[pl.ds(j * L, L)] * s0
        acc_v[pl.ds(D + j * L, L)] = acc_v[pl.ds(D + j * L, L)] * s1
    pltpu.sync_copy(acc_v, out_hbm.at[pl.ds(wid * SEG_PER_W * D, SEG_PER_W * D)])


@jax.jit
def _pool(x, bounds):
    mesh = plsc.VectorSubcoreMesh(core_axis_name="c", subcore_axis_name="s",
                                  num_cores=NC, num_subcores=NS)
    return pl.kernel(
        _body,
        out_type=jax.ShapeDtypeStruct((NUM_SEG * D,), jnp.float32),
        mesh=mesh,
        scratch_types=[
            pltpu.VMEM((80,), jnp.int32),
            pltpu.VMEM((CH, D), jnp.float32),
            pltpu.VMEM((SEG_PER_W * D,), jnp.float32),
        ],
    )(x, bounds)


def kernel(x, batch):
    bounds = jnp.searchsorted(batch, jnp.arange(65, dtype=batch.dtype),
                              side="left").astype(jnp.int32)
    bounds = jnp.concatenate(
        [bounds, jnp.full((15,), x.shape[0], jnp.int32)])
    return _pool(x, bounds).reshape(NUM_SEG, D)


# speedup vs baseline: 6.4333x; 2.2069x over previous
import jax
import jax.numpy as jnp
from jax import lax
from jax.experimental import pallas as pl
from jax.experimental.pallas import tpu as pltpu
from jax.experimental.pallas import tpu_sc as plsc

NC = 2
NS = 16
NW = NC * NS
L = 16
NUM_SEG = 64
SEG_PER_W = NUM_SEG // NW
N_ROWS = 50000
D = 256
NJ = D // L
CH = 128


def _body(x_hbm, bounds_hbm, out_hbm, bounds_v, buf0_v, buf1_v, acc_v,
          sem0, sem1):
    cid = lax.axis_index("c")
    sid = lax.axis_index("s")
    wid = sid * NC + cid

    pltpu.sync_copy(bounds_hbm, bounds_v)

    bv = bounds_v[pl.ds(SEG_PER_W * wid, L)]
    a0 = bv[0]
    m = bv[1]
    b1 = bv[2]

    a8 = lax.div(a0, 8) * 8
    nch = lax.div(b1 - a8 + (CH - 1), CH)
    npairs = lax.div(nch + 1, 2)

    def start(c, buf):
        base = a8 + c * CH
        clamped = pl.multiple_of(jnp.minimum(base, N_ROWS - CH), 8)
        sem = sem0 if buf is buf0_v else sem1
        pltpu.make_async_copy(x_hbm.at[pl.ds(clamped, CH)], buf, sem).start()

    def wait(buf):
        sem = sem0 if buf is buf0_v else sem1
        pltpu.make_async_copy(x_hbm.at[pl.ds(0, CH)], buf, sem).wait()

    zero = jnp.zeros((L,), jnp.float32)
    accs = (tuple(zero for _ in range(NJ)), tuple(zero for _ in range(NJ)))

    def compute(c, buf, accs):
        base = a8 + c * CH
        clamped = jnp.minimum(base, N_ROWS - CH)
        lo0 = jnp.maximum(base, a0) - clamped
        hi0 = jnp.maximum(jnp.minimum(m, clamped + CH) - clamped, lo0)
        lo1 = jnp.maximum(base, m) - clamped
        hi1 = jnp.maximum(jnp.minimum(b1, clamped + CH) - clamped, lo1)

        def row_body(r, acc):
            return tuple(acc[j] + buf[r, pl.ds(j * L, L)] for j in range(NJ))

        return (lax.fori_loop(lo0, hi0, row_body, accs[0]),
                lax.fori_loop(lo1, hi1, row_body, accs[1]))

    start(0, buf0_v)

    def pair_body(g, accs):
        c0 = 2 * g
        start(c0 + 1, buf1_v)
        wait(buf0_v)
        accs = compute(c0, buf0_v, accs)
        start(c0 + 2, buf0_v)
        wait(buf1_v)
        accs = compute(c0 + 1, buf1_v, accs)
        return accs

    accs = lax.fori_loop(0, npairs, pair_body, accs)
    wait(buf0_v)

    one = jnp.ones((L,), jnp.float32)
    n0 = one * (m - a0).astype(jnp.float32)
    n1 = one * (b1 - m).astype(jnp.float32)
    s0 = jnp.where(n0 > 0.0, one / jnp.maximum(n0, one), 0.0)
    s1 = jnp.where(n1 > 0.0, one / jnp.maximum(n1, one), 0.0)
    for j in range(NJ):
        acc_v[pl.ds(j * L, L)] = accs[0][j] * s0
        acc_v[pl.ds(D + j * L, L)] = accs[1][j] * s1
    pltpu.sync_copy(acc_v, out_hbm.at[pl.ds(wid * SEG_PER_W * D, SEG_PER_W * D)])


@jax.jit
def _pool(x, bounds):
    mesh = plsc.VectorSubcoreMesh(core_axis_name="c", subcore_axis_name="s",
                                  num_cores=NC, num_subcores=NS)
    return pl.kernel(
        _body,
        out_type=jax.ShapeDtypeStruct((NUM_SEG * D,), jnp.float32),
        mesh=mesh,
        scratch_types=[
            pltpu.VMEM((80,), jnp.int32),
            pltpu.VMEM((CH, D), jnp.float32),
            pltpu.VMEM((CH, D), jnp.float32),
            pltpu.VMEM((SEG_PER_W * D,), jnp.float32),
            pltpu.SemaphoreType.DMA,
            pltpu.SemaphoreType.DMA,
        ],
    )(x, bounds)


def kernel(x, batch):
    bounds = jnp.searchsorted(batch, jnp.arange(65, dtype=batch.dtype),
                              side="left").astype(jnp.int32)
    bounds = jnp.concatenate(
        [bounds, jnp.full((15,), x.shape[0], jnp.int32)])
    return _pool(x, bounds).reshape(NUM_SEG, D)


# speedup vs baseline: 9.5794x; 1.4890x over previous
import jax
import jax.numpy as jnp
from jax import lax
from jax.experimental import pallas as pl
from jax.experimental.pallas import tpu as pltpu
from jax.experimental.pallas import tpu_sc as plsc

NC = 2
NS = 16
NW = NC * NS
L = 16
NUM_SEG = 64
SEG_PER_W = NUM_SEG // NW
N_ROWS = 50000
D = 256
NJ = D // L
CH = 128


def _body(x_hbm, bounds_hbm, out_hbm, bounds_v, buf0_v, buf1_v, acc_v,
          sem0, sem1):
    cid = lax.axis_index("c")
    sid = lax.axis_index("s")
    wid = sid * NC + cid

    pltpu.sync_copy(bounds_hbm, bounds_v)

    bv = bounds_v[pl.ds(SEG_PER_W * wid, L)]
    a0 = bv[0]
    m = bv[1]
    b1 = bv[2]

    a8 = lax.div(a0, 8) * 8
    nch = lax.div(b1 - a8 + (CH - 1), CH)
    npairs = lax.div(nch + 1, 2)

    def start(c, buf):
        base = a8 + c * CH
        clamped = pl.multiple_of(jnp.minimum(base, N_ROWS - CH), 8)
        sem = sem0 if buf is buf0_v else sem1
        pltpu.make_async_copy(x_hbm.at[pl.ds(clamped, CH)], buf, sem).start()

    def wait(buf):
        sem = sem0 if buf is buf0_v else sem1
        pltpu.make_async_copy(x_hbm.at[pl.ds(0, CH)], buf, sem).wait()

    zero = jnp.zeros((L,), jnp.float32)
    accs = (tuple(zero for _ in range(NJ)), tuple(zero for _ in range(NJ)))

    def compute(c, buf, accs):
        base = a8 + c * CH
        clamped = jnp.minimum(base, N_ROWS - CH)
        lo0 = jnp.maximum(base, a0) - clamped
        hi0 = jnp.maximum(jnp.minimum(m, clamped + CH) - clamped, lo0)
        lo1 = jnp.maximum(base, m) - clamped
        hi1 = jnp.maximum(jnp.minimum(b1, clamped + CH) - clamped, lo1)

        def row_body(r, acc):
            return tuple(acc[j] + buf[r, pl.ds(j * L, L)] for j in range(NJ))

        return (lax.fori_loop(lo0, hi0, row_body, accs[0]),
                lax.fori_loop(lo1, hi1, row_body, accs[1]))

    start(0, buf0_v)

    def pair_body(g, accs):
        c0 = 2 * g
        start(c0 + 1, buf1_v)
        wait(buf0_v)
        accs = compute(c0, buf0_v, accs)
        start(c0 + 2, buf0_v)
        wait(buf1_v)
        accs = compute(c0 + 1, buf1_v, accs)
        return accs

    accs = lax.fori_loop(0, npairs, pair_body, accs)
    wait(buf0_v)

    one = jnp.ones((L,), jnp.float32)
    n0 = one * (m - a0).astype(jnp.float32)
    n1 = one * (b1 - m).astype(jnp.float32)
    s0 = jnp.where(n0 > 0.0, one / jnp.maximum(n0, one), 0.0)
    s1 = jnp.where(n1 > 0.0, one / jnp.maximum(n1, one), 0.0)
    for j in range(NJ):
        acc_v[pl.ds(j * L, L)] = accs[0][j] * s0
        acc_v[pl.ds(D + j * L, L)] = accs[1][j] * s1
    pltpu.sync_copy(acc_v, out_hbm.at[pl.ds(wid * SEG_PER_W * D, SEG_PER_W * D)])


@jax.jit
def _pool(x, bounds):
    mesh = plsc.VectorSubcoreMesh(core_axis_name="c", subcore_axis_name="s",
                                  num_cores=NC, num_subcores=NS)
    return pl.kernel(
        _body,
        out_type=jax.ShapeDtypeStruct((NUM_SEG * D,), jnp.float32),
        mesh=mesh,
        scratch_types=[
            pltpu.VMEM((80,), jnp.int32),
            pltpu.VMEM((CH, D), jnp.float32),
            pltpu.VMEM((CH, D), jnp.float32),
            pltpu.VMEM((SEG_PER_W * D,), jnp.float32),
            pltpu.SemaphoreType.DMA,
            pltpu.SemaphoreType.DMA,
        ],
    )(x, bounds)


def kernel(x, batch):
    seg = jnp.arange(NUM_SEG, dtype=batch.dtype)
    counts = jnp.sum((batch[:, None] == seg[None, :]).astype(jnp.int32),
                     axis=0)
    bounds = jnp.concatenate(
        [jnp.zeros((1,), jnp.int32), jnp.cumsum(counts),
         jnp.full((15,), x.shape[0], jnp.int32)]).astype(jnp.int32)
    return _pool(x, bounds).reshape(NUM_SEG, D)
